# baseline (speedup 1.0000x reference)
import jax
import jax.numpy as jnp
from jax.experimental import pallas as pl

SEQ = 200
DIM = 64
FEAT = SEQ * DIM


def _add_kernel(ex_ref, cat_ref, pos_ref, out_ref):
    out_ref[:] = ex_ref[:] + cat_ref[:] + pos_ref[:][None]


def kernel(exercises, categories, position_embed):
    B = exercises.shape[0]
    BB = 64
    out = pl.pallas_call(
        _add_kernel,
        grid=(B // BB,),
        in_specs=[
            pl.BlockSpec((BB, SEQ, DIM), lambda i: (i, 0, 0)),
            pl.BlockSpec((BB, SEQ, DIM), lambda i: (i, 0, 0)),
            pl.BlockSpec((SEQ, DIM), lambda i: (0, 0)),
        ],
        out_specs=pl.BlockSpec((BB, SEQ, DIM), lambda i: (i, 0, 0)),
        out_shape=jax.ShapeDtypeStruct((B, SEQ, DIM), jnp.float32),
    )(exercises, categories, position_embed)
    return out
